# baseline (speedup 1.0000x reference)
import functools

import jax
import jax.numpy as jnp
from jax import lax
from jax.experimental import pallas as pl
from jax.experimental.pallas import tpu as pltpu
from jax.experimental.pallas import tpu_sc as plsc

_UP = 4
_BIG = 3.0e38


def _linear_body(x_ref, w_ref, b_ref, o_ref, *, act):
    y = jnp.dot(x_ref[...], w_ref[...], preferred_element_type=jnp.float32)
    y = y + b_ref[...]
    if act:
        y = jnp.maximum(y, 0.0)
    o_ref[...] = y


def _linear(x, w, b, act=False):
    n = x.shape[0]
    do = w.shape[1]
    return pl.pallas_call(
        functools.partial(_linear_body, act=act),
        out_shape=jax.ShapeDtypeStruct((n, do), jnp.float32),
    )(x, w, b.reshape(1, do))


def _mlp2_body(x_ref, w1_ref, b1_ref, w2_ref, b2_ref, res_ref, o_ref, *, scale):
    h = jnp.dot(x_ref[...], w1_ref[...], preferred_element_type=jnp.float32)
    h = jnp.maximum(h + b1_ref[...], 0.0)
    y = jnp.dot(h, w2_ref[...], preferred_element_type=jnp.float32) + b2_ref[...]
    o_ref[...] = res_ref[...] + scale * y


def _mlp2(x, w1, b1, w2, b2, res, scale):
    n = x.shape[0]
    do = w2.shape[1]
    return pl.pallas_call(
        functools.partial(_mlp2_body, scale=scale),
        out_shape=jax.ShapeDtypeStruct((n, do), jnp.float32),
    )(x, w1, b1.reshape(1, -1), w2, b2.reshape(1, -1), res)


def _fo_body(x_ref, xyz_ref, w1_ref, b1_ref, w2_ref, b2_ref, w3_ref, b3_ref,
             wr_ref, br_ref, o_ref, ab_ref):
    x = x_ref[...]
    h = jnp.dot(x, w1_ref[...], preferred_element_type=jnp.float32)
    h = jnp.maximum(h + b1_ref[...], 0.0)
    h = jnp.dot(h, w2_ref[...], preferred_element_type=jnp.float32)
    h = jnp.maximum(h + b2_ref[...], 0.0)
    y = jnp.dot(h, w3_ref[...], preferred_element_type=jnp.float32) + b3_ref[...]
    xyz_new = xyz_ref[...] + y
    o_ref[...] = xyz_new
    xr = jnp.concatenate([x[:, 3:], xyz_new], axis=1)
    ab_ref[...] = jnp.dot(xr, wr_ref[...],
                          preferred_element_type=jnp.float32) + br_ref[...]


def _fo_mlp(x, xyz, w1, b1, w2, b2, w3, b3, wr, br):
    n = x.shape[0]
    return pl.pallas_call(
        _fo_body,
        out_shape=[
            jax.ShapeDtypeStruct((n, 3), jnp.float32),
            jax.ShapeDtypeStruct((n, wr.shape[1]), jnp.float32),
        ],
    )(x, xyz, w1, b1.reshape(1, -1), w2, b2.reshape(1, -1), w3,
      b3.reshape(1, -1), wr, br.reshape(1, -1))


def _knn_body(xb_ref, x_ref, o_ref, *, k, n, rb, exclude_self):
    i = pl.program_id(0)
    x = x_ref[...]
    xb = xb_ref[...]
    sq = jnp.sum(x * x, axis=1)
    sqb = jnp.sum(xb * xb, axis=1)
    d = sqb[:, None] - 2.0 * lax.dot_general(
        xb, x, (((1,), (1,)), ((), ())), preferred_element_type=jnp.float32)
    d = d + sq[None, :]
    col = lax.broadcasted_iota(jnp.int32, (rb, n), 1)
    if exclude_self:
        row = i * rb + lax.broadcasted_iota(jnp.int32, (rb, n), 0)
        d = jnp.where(col == row, _BIG, d)
    idx_mat = jnp.zeros((rb, 128), jnp.int32)
    colj = lax.broadcasted_iota(jnp.int32, (rb, 128), 1)
    for j in range(k):
        m = jnp.min(d, axis=1, keepdims=True)
        hit = d == m
        idxj = jnp.min(jnp.where(hit, col, n), axis=1)
        idx_mat = jnp.where(colj == j, idxj[:, None], idx_mat)
        if j < k - 1:
            d = jnp.where(hit, _BIG, d)
    o_ref[...] = idx_mat


def _knn(x, k, exclude_self=False):
    n, dd = x.shape
    rb = 512 if n >= 8192 else 256
    grid = (n // rb,)
    return pl.pallas_call(
        functools.partial(_knn_body, k=k, n=n, rb=rb, exclude_self=exclude_self),
        grid=grid,
        in_specs=[
            pl.BlockSpec((rb, dd), lambda i: (i, 0)),
            pl.BlockSpec((n, dd), lambda i: (0, 0)),
        ],
        out_specs=pl.BlockSpec((rb, 128), lambda i: (i, 0)),
        out_shape=jax.ShapeDtypeStruct((n, 128), jnp.int32),
    )(x, x)


def _ecmax_body(a_ref, g_ref, w2_ref, b2_ref, res_ref, o_ref, *, k, dh):
    a = a_ref[...]
    acc = None
    for j in range(k):
        h = jnp.maximum(a + g_ref[:, j * dh:(j + 1) * dh], 0.0)
        h = jnp.dot(h, w2_ref[...], preferred_element_type=jnp.float32)
        acc = h if acc is None else jnp.maximum(acc, h)
    o_ref[...] = acc + b2_ref[...] + res_ref[...]


def _ecmax_lin_body(a_ref, g_ref, w2_ref, b2_ref, wn_ref, bn_ref, o_ref,
                    ab_ref, *, k, dh):
    a = a_ref[...]
    acc = None
    for j in range(k):
        h = jnp.maximum(a + g_ref[:, j * dh:(j + 1) * dh], 0.0)
        h = jnp.dot(h, w2_ref[...], preferred_element_type=jnp.float32)
        acc = h if acc is None else jnp.maximum(acc, h)
    out = acc + b2_ref[...]
    o_ref[...] = out
    ab_ref[...] = jnp.dot(out, wn_ref[...],
                          preferred_element_type=jnp.float32) + bn_ref[...]


def _ecmax(a, g, w2, b2, res):
    n, dh = a.shape
    k = g.shape[1] // dh
    do = w2.shape[1]
    rb = 256
    grid = (n // rb,)
    return pl.pallas_call(
        functools.partial(_ecmax_body, k=k, dh=dh),
        grid=grid,
        in_specs=[
            pl.BlockSpec((rb, dh), lambda i: (i, 0)),
            pl.BlockSpec((rb, k * dh), lambda i: (i, 0)),
            pl.BlockSpec((dh, do), lambda i: (0, 0)),
            pl.BlockSpec((1, do), lambda i: (0, 0)),
            pl.BlockSpec((rb, do), lambda i: (i, 0)),
        ],
        out_specs=pl.BlockSpec((rb, do), lambda i: (i, 0)),
        out_shape=jax.ShapeDtypeStruct((n, do), jnp.float32),
    )(a, g, w2, b2.reshape(1, do), res)


def _ecmax_lin(a, g, w2, b2, wn, bn):
    n, dh = a.shape
    k = g.shape[1] // dh
    do = w2.shape[1]
    dn = wn.shape[1]
    rb = 256
    grid = (n // rb,)
    return pl.pallas_call(
        functools.partial(_ecmax_lin_body, k=k, dh=dh),
        grid=grid,
        in_specs=[
            pl.BlockSpec((rb, dh), lambda i: (i, 0)),
            pl.BlockSpec((rb, k * dh), lambda i: (i, 0)),
            pl.BlockSpec((dh, do), lambda i: (0, 0)),
            pl.BlockSpec((1, do), lambda i: (0, 0)),
            pl.BlockSpec((do, dn), lambda i: (0, 0)),
            pl.BlockSpec((1, dn), lambda i: (0, 0)),
        ],
        out_specs=[
            pl.BlockSpec((rb, do), lambda i: (i, 0)),
            pl.BlockSpec((rb, dn), lambda i: (i, 0)),
        ],
        out_shape=[
            jax.ShapeDtypeStruct((n, do), jnp.float32),
            jax.ShapeDtypeStruct((n, dn), jnp.float32),
        ],
    )(a, g, w2, b2.reshape(1, do), wn, bn.reshape(1, dn))


def _sc_gather(table, idx):
    m = idx.shape[0]
    dd = table.shape[1]
    nw = 32
    per_w = m // nw
    max_sub = 32768 // dd
    sub = max_sub if per_w % max_sub == 0 else per_w
    nch = per_w // sub
    mesh = plsc.VectorSubcoreMesh(core_axis_name="c", subcore_axis_name="s")

    @functools.partial(
        pl.kernel,
        out_type=jax.ShapeDtypeStruct((m, dd), jnp.float32),
        mesh=mesh,
        compiler_params=pltpu.CompilerParams(use_tc_tiling_on_sc=False),
        scratch_types=[
            pltpu.VMEM((per_w,), jnp.int32),
            pltpu.VMEM((sub, dd), jnp.float32),
            pltpu.VMEM((sub, dd), jnp.float32),
            pltpu.SemaphoreType.DMA,
            pltpu.SemaphoreType.DMA,
        ],
    )
    def gk(table_hbm, idx_hbm, out_hbm, idx_v, rows_v0, rows_v1, sem0, sem1):
        wid = lax.axis_index("s") * 2 + lax.axis_index("c")
        base = wid * per_w
        pltpu.sync_copy(idx_hbm.at[pl.ds(base, per_w)], idx_v)
        rows = (rows_v0, rows_v1)
        sems = (sem0, sem1)
        copies = []
        for c in range(nch):
            copies.append(pltpu.async_copy(
                table_hbm.at[idx_v.at[pl.ds(c * sub, sub)]], rows[c % 2],
                sems[c % 2]))
            if c >= 1:
                copies[c - 1].wait()
                pltpu.sync_copy(rows[(c - 1) % 2],
                                out_hbm.at[pl.ds(base + (c - 1) * sub, sub)])
        copies[nch - 1].wait()
        pltpu.sync_copy(rows[(nch - 1) % 2],
                        out_hbm.at[pl.ds(base + (nch - 1) * sub, sub)])

    return gk(table, idx)


def _knn_flat(x, k, exclude_self=False):
    idx = _knn(x, k, exclude_self)
    return idx[:, :k].reshape(-1)


def _ec_wprep(w1, b1):
    d = w1.shape[0] // 2
    w1a, w1b = w1[:d], w1[d:]
    wcat = jnp.concatenate([w1a - w1b, w1b], axis=1)
    bcat = jnp.concatenate([b1, jnp.zeros_like(b1)])
    return wcat, bcat


def _dynconv(x, p, k1, k2):
    n = x.shape[0]
    dh1 = p['dc1_W1'].shape[1]
    dh2 = p['dc2_W1'].shape[1]
    wcat1, bcat1 = _ec_wprep(p['dc1_W1'], p['dc1_b1'])
    wcat2, bcat2 = _ec_wprep(p['dc2_W1'], p['dc2_b1'])
    idx1 = _knn_flat(x, k1)
    ab1 = _linear(x, wcat1, bcat1)
    g1 = _sc_gather(ab1[:, dh1:], idx1).reshape(n, k1 * dh1)
    h, ab2 = _ecmax_lin(ab1[:, :dh1], g1, p['dc1_W2'], p['dc1_b2'],
                        wcat2, bcat2)
    idx2 = _knn_flat(h, k2)
    g2 = _sc_gather(ab2[:, dh2:], idx2).reshape(n, k2 * dh2)
    res = jnp.zeros((n, p['dc2_W2'].shape[1]), jnp.float32)
    return _ecmax(ab2[:, :dh2], g2, p['dc2_W2'], p['dc2_b2'], res)


def kernel(ctx_xyz, ctx_tokens, pred_tokens, mask_id, params):
    p = params
    bb, mm, pp, c = pred_tokens.shape
    n_c = bb * pp
    n_t = n_c * _UP

    pred_tok_m = jnp.take(pred_tokens, mask_id, axis=1)
    distinct = pred_tok_m.reshape(n_c, c)
    ctx_tok_f = ctx_tokens.reshape(n_c, c)
    ctx_xyz_f = ctx_xyz.reshape(n_c, 3)

    ctx_feat = _dynconv(ctx_tok_f, p, 16, 8)
    ctx_out = _mlp2(ctx_feat, p['cd_W1'], p['cd_b1'], p['cd_W2'], p['cd_b2'],
                    ctx_xyz_f, 0.05)

    seed = _mlp2(distinct, p['lat_W1'], p['lat_b1'], p['lat_W2'], p['lat_b2'],
                 jnp.zeros((n_c, 3), jnp.float32), 1.0)
    feat_t = _dynconv(distinct, p, 4, 2)
    tgt_feat = jnp.broadcast_to(
        feat_t.reshape(bb, pp, 1, feat_t.shape[1]),
        (bb, pp, _UP, feat_t.shape[1])).reshape(n_t, feat_t.shape[1])
    seed_rep = jnp.broadcast_to(
        seed.reshape(bb, pp, 1, 3), (bb, pp, _UP, 3)).reshape(bb, pp * _UP, 3)
    noise = jax.random.normal(jax.random.key(777), (bb, pp * _UP, 3),
                              dtype=jnp.float32) * 0.02
    tgt_xyz_f = (seed_rep + noise).reshape(n_t, 3)

    x_fold = jnp.concatenate([tgt_xyz_f, tgt_feat], axis=1)
    wcat_r, bcat_r = _ec_wprep(p['rf_W1'], p['rf_b1'])
    tgt_xyz_f, ab_r = _fo_mlp(x_fold, tgt_xyz_f, p['fo_W1'], p['fo_b1'],
                              p['fo_W2'], p['fo_b2'], p['fo_W3'], p['fo_b3'],
                              wcat_r, bcat_r)

    idx_f = _knn_flat(tgt_xyz_f, 16, exclude_self=True)
    dh_r = p['rf_W1'].shape[1]
    g_r = _sc_gather(ab_r[:, dh_r:], idx_f).reshape(n_t, 16 * dh_r)
    tgt_out = _ecmax(ab_r[:, :dh_r], g_r, p['rf_W2'], p['rf_b2'], tgt_xyz_f)

    return jnp.concatenate([ctx_out, tgt_out], axis=0)


# speedup vs baseline: 1.1137x; 1.1137x over previous
import functools

import jax
import jax.numpy as jnp
from jax import lax
from jax.experimental import pallas as pl
from jax.experimental.pallas import tpu as pltpu
from jax.experimental.pallas import tpu_sc as plsc

_UP = 4
_BIG = 3.0e38


def _fo_body(x_ref, xyz_ref, w1_ref, b1_ref, w2_ref, b2_ref, w3_ref, b3_ref,
             wr_ref, br_ref, o_ref, ab_ref):
    x = x_ref[...]
    h = jnp.dot(x, w1_ref[...], preferred_element_type=jnp.float32)
    h = jnp.maximum(h + b1_ref[...], 0.0)
    h = jnp.dot(h, w2_ref[...], preferred_element_type=jnp.float32)
    h = jnp.maximum(h + b2_ref[...], 0.0)
    y = jnp.dot(h, w3_ref[...], preferred_element_type=jnp.float32) + b3_ref[...]
    xyz_new = xyz_ref[...] + y
    o_ref[...] = xyz_new
    xr = jnp.concatenate([x[:, 3:], xyz_new], axis=1)
    ab_ref[...] = jnp.dot(xr, wr_ref[...],
                          preferred_element_type=jnp.float32) + br_ref[...]


def _fo_mlp(x, xyz, w1, b1, w2, b2, w3, b3, wr, br):
    n = x.shape[0]
    return pl.pallas_call(
        _fo_body,
        out_shape=[
            jax.ShapeDtypeStruct((n, 3), jnp.float32),
            jax.ShapeDtypeStruct((n, wr.shape[1]), jnp.float32),
        ],
    )(x, xyz, w1, b1.reshape(1, -1), w2, b2.reshape(1, -1), w3,
      b3.reshape(1, -1), wr, br.reshape(1, -1))


def _knn_body(xb_ref, x_ref, *refs, k, n, rb, exclude_self, fuse_ab, fuse_mlp):
    i = pl.program_id(0)
    x = x_ref[...]
    xb = xb_ref[...]
    refs = list(refs)
    if fuse_ab:
        wc_ref, bc_ref = refs[0], refs[1]
        refs = refs[2:]
    if fuse_mlp:
        w1_ref, b1_ref, w2_ref, b2_ref = refs[:4]
        refs = refs[4:]
    o_ref = refs[0]
    if fuse_ab:
        ab_ref = refs[1]
        ab_ref[...] = jnp.dot(xb, wc_ref[...],
                              preferred_element_type=jnp.float32) + bc_ref[...]
    if fuse_mlp:
        s_ref = refs[2]
        hh = jnp.dot(xb, w1_ref[...], preferred_element_type=jnp.float32)
        hh = jnp.maximum(hh + b1_ref[...], 0.0)
        s_ref[...] = jnp.dot(hh, w2_ref[...],
                             preferred_element_type=jnp.float32) + b2_ref[...]
    sq = jnp.sum(x * x, axis=1)
    sqb = jnp.sum(xb * xb, axis=1)
    d = sqb[:, None] - 2.0 * lax.dot_general(
        xb, x, (((1,), (1,)), ((), ())), preferred_element_type=jnp.float32)
    d = d + sq[None, :]
    col = lax.broadcasted_iota(jnp.int32, (rb, n), 1)
    if exclude_self:
        row = i * rb + lax.broadcasted_iota(jnp.int32, (rb, n), 0)
        d = jnp.where(col == row, _BIG, d)
    idx_mat = jnp.zeros((rb, 128), jnp.int32)
    colj = lax.broadcasted_iota(jnp.int32, (rb, 128), 1)
    for j in range(k):
        m = jnp.min(d, axis=1, keepdims=True)
        hit = d == m
        idxj = jnp.min(jnp.where(hit, col, n), axis=1)
        idx_mat = jnp.where(colj == j, idxj[:, None], idx_mat)
        if j < k - 1:
            d = jnp.where(hit, _BIG, d)
    o_ref[...] = idx_mat


def _knn(x, k, exclude_self=False, ab_w=None, mlp_w=None):
    n, dd = x.shape
    rb = 256
    grid = (n // rb,)
    in_specs = [
        pl.BlockSpec((rb, dd), lambda i: (i, 0)),
        pl.BlockSpec((n, dd), lambda i: (0, 0)),
    ]
    args = [x, x]
    out_specs = [pl.BlockSpec((rb, 128), lambda i: (i, 0))]
    out_shape = [jax.ShapeDtypeStruct((n, 128), jnp.int32)]
    if ab_w is not None:
        wcat, bcat = ab_w
        in_specs += [pl.BlockSpec(wcat.shape, lambda i: (0, 0)),
                     pl.BlockSpec((1, bcat.shape[0]), lambda i: (0, 0))]
        args += [wcat, bcat.reshape(1, -1)]
        out_specs.append(pl.BlockSpec((rb, wcat.shape[1]), lambda i: (i, 0)))
        out_shape.append(jax.ShapeDtypeStruct((n, wcat.shape[1]), jnp.float32))
    if mlp_w is not None:
        w1, b1, w2, b2 = mlp_w
        in_specs += [pl.BlockSpec(w1.shape, lambda i: (0, 0)),
                     pl.BlockSpec((1, b1.shape[0]), lambda i: (0, 0)),
                     pl.BlockSpec(w2.shape, lambda i: (0, 0)),
                     pl.BlockSpec((1, b2.shape[0]), lambda i: (0, 0))]
        args += [w1, b1.reshape(1, -1), w2, b2.reshape(1, -1)]
        out_specs.append(pl.BlockSpec((rb, w2.shape[1]), lambda i: (i, 0)))
        out_shape.append(jax.ShapeDtypeStruct((n, w2.shape[1]), jnp.float32))
    res = pl.pallas_call(
        functools.partial(_knn_body, k=k, n=n, rb=rb, exclude_self=exclude_self,
                          fuse_ab=ab_w is not None, fuse_mlp=mlp_w is not None),
        grid=grid,
        in_specs=in_specs,
        out_specs=out_specs if len(out_specs) > 1 else out_specs[0],
        out_shape=out_shape if len(out_shape) > 1 else out_shape[0],
    )(*args)
    return res if isinstance(res, (list, tuple)) else (res,)


def _ecmax_body(a_ref, g_ref, w2_ref, b2_ref, res_ref, o_ref, *, k, dh):
    a = a_ref[...]
    acc = None
    for j in range(k):
        h = jnp.maximum(a + g_ref[:, j * dh:(j + 1) * dh], 0.0)
        h = jnp.dot(h, w2_ref[...], preferred_element_type=jnp.float32)
        acc = h if acc is None else jnp.maximum(acc, h)
    o_ref[...] = acc + b2_ref[...] + res_ref[...]


def _ecmax_lin_body(a_ref, g_ref, w2_ref, b2_ref, wn_ref, bn_ref, o_ref,
                    ab_ref, *, k, dh):
    a = a_ref[...]
    acc = None
    for j in range(k):
        h = jnp.maximum(a + g_ref[:, j * dh:(j + 1) * dh], 0.0)
        h = jnp.dot(h, w2_ref[...], preferred_element_type=jnp.float32)
        acc = h if acc is None else jnp.maximum(acc, h)
    out = acc + b2_ref[...]
    o_ref[...] = out
    ab_ref[...] = jnp.dot(out, wn_ref[...],
                          preferred_element_type=jnp.float32) + bn_ref[...]


def _ecmax_mlp2_body(a_ref, g_ref, w2_ref, b2_ref, mw1_ref, mb1_ref, mw2_ref,
                     mb2_ref, res_ref, o_ref, *, k, dh, scale):
    a = a_ref[...]
    acc = None
    for j in range(k):
        h = jnp.maximum(a + g_ref[:, j * dh:(j + 1) * dh], 0.0)
        h = jnp.dot(h, w2_ref[...], preferred_element_type=jnp.float32)
        acc = h if acc is None else jnp.maximum(acc, h)
    feat = acc + b2_ref[...]
    hh = jnp.dot(feat, mw1_ref[...], preferred_element_type=jnp.float32)
    hh = jnp.maximum(hh + mb1_ref[...], 0.0)
    y = jnp.dot(hh, mw2_ref[...], preferred_element_type=jnp.float32)
    o_ref[...] = res_ref[...] + scale * (y + mb2_ref[...])


def _ecmax_mlp2(a, g, w2, b2, mw1, mb1, mw2, mb2, res, scale):
    n, dh = a.shape
    k = g.shape[1] // dh
    do = w2.shape[1]
    dm = mw2.shape[1]
    rb = 256
    grid = (n // rb,)
    return pl.pallas_call(
        functools.partial(_ecmax_mlp2_body, k=k, dh=dh, scale=scale),
        grid=grid,
        in_specs=[
            pl.BlockSpec((rb, dh), lambda i: (i, 0)),
            pl.BlockSpec((rb, k * dh), lambda i: (i, 0)),
            pl.BlockSpec((dh, do), lambda i: (0, 0)),
            pl.BlockSpec((1, do), lambda i: (0, 0)),
            pl.BlockSpec(mw1.shape, lambda i: (0, 0)),
            pl.BlockSpec((1, mb1.shape[0]), lambda i: (0, 0)),
            pl.BlockSpec(mw2.shape, lambda i: (0, 0)),
            pl.BlockSpec((1, dm), lambda i: (0, 0)),
            pl.BlockSpec((rb, dm), lambda i: (i, 0)),
        ],
        out_specs=pl.BlockSpec((rb, dm), lambda i: (i, 0)),
        out_shape=jax.ShapeDtypeStruct((n, dm), jnp.float32),
    )(a, g, w2, b2.reshape(1, do), mw1, mb1.reshape(1, -1), mw2,
      mb2.reshape(1, dm), res)


def _ecmax(a, g, w2, b2, res):
    n, dh = a.shape
    k = g.shape[1] // dh
    do = w2.shape[1]
    rb = 256
    grid = (n // rb,)
    return pl.pallas_call(
        functools.partial(_ecmax_body, k=k, dh=dh),
        grid=grid,
        in_specs=[
            pl.BlockSpec((rb, dh), lambda i: (i, 0)),
            pl.BlockSpec((rb, k * dh), lambda i: (i, 0)),
            pl.BlockSpec((dh, do), lambda i: (0, 0)),
            pl.BlockSpec((1, do), lambda i: (0, 0)),
            pl.BlockSpec((rb, do), lambda i: (i, 0)),
        ],
        out_specs=pl.BlockSpec((rb, do), lambda i: (i, 0)),
        out_shape=jax.ShapeDtypeStruct((n, do), jnp.float32),
    )(a, g, w2, b2.reshape(1, do), res)


def _ecmax_lin(a, g, w2, b2, wn, bn):
    n, dh = a.shape
    k = g.shape[1] // dh
    do = w2.shape[1]
    dn = wn.shape[1]
    rb = 256
    grid = (n // rb,)
    return pl.pallas_call(
        functools.partial(_ecmax_lin_body, k=k, dh=dh),
        grid=grid,
        in_specs=[
            pl.BlockSpec((rb, dh), lambda i: (i, 0)),
            pl.BlockSpec((rb, k * dh), lambda i: (i, 0)),
            pl.BlockSpec((dh, do), lambda i: (0, 0)),
            pl.BlockSpec((1, do), lambda i: (0, 0)),
            pl.BlockSpec((do, dn), lambda i: (0, 0)),
            pl.BlockSpec((1, dn), lambda i: (0, 0)),
        ],
        out_specs=[
            pl.BlockSpec((rb, do), lambda i: (i, 0)),
            pl.BlockSpec((rb, dn), lambda i: (i, 0)),
        ],
        out_shape=[
            jax.ShapeDtypeStruct((n, do), jnp.float32),
            jax.ShapeDtypeStruct((n, dn), jnp.float32),
        ],
    )(a, g, w2, b2.reshape(1, do), wn, bn.reshape(1, dn))


def _sc_gather(table, idx):
    m = idx.shape[0]
    dd = table.shape[1]
    nw = 32
    per_w = m // nw
    max_sub = 32768 // dd
    sub = max_sub if per_w % max_sub == 0 else per_w
    nch = per_w // sub
    mesh = plsc.VectorSubcoreMesh(core_axis_name="c", subcore_axis_name="s")

    @functools.partial(
        pl.kernel,
        out_type=jax.ShapeDtypeStruct((m, dd), jnp.float32),
        mesh=mesh,
        compiler_params=pltpu.CompilerParams(use_tc_tiling_on_sc=False),
        scratch_types=[
            pltpu.VMEM((per_w,), jnp.int32),
            pltpu.VMEM((sub, dd), jnp.float32),
            pltpu.VMEM((sub, dd), jnp.float32),
            pltpu.SemaphoreType.DMA,
            pltpu.SemaphoreType.DMA,
        ],
    )
    def gk(table_hbm, idx_hbm, out_hbm, idx_v, rows_v0, rows_v1, sem0, sem1):
        wid = lax.axis_index("s") * 2 + lax.axis_index("c")
        base = wid * per_w
        pltpu.sync_copy(idx_hbm.at[pl.ds(base, per_w)], idx_v)
        rows = (rows_v0, rows_v1)
        sems = (sem0, sem1)
        copies = []
        for c in range(nch):
            copies.append(pltpu.async_copy(
                table_hbm.at[idx_v.at[pl.ds(c * sub, sub)]], rows[c % 2],
                sems[c % 2]))
            if c >= 1:
                copies[c - 1].wait()
                pltpu.sync_copy(rows[(c - 1) % 2],
                                out_hbm.at[pl.ds(base + (c - 1) * sub, sub)])
        copies[nch - 1].wait()
        pltpu.sync_copy(rows[(nch - 1) % 2],
                        out_hbm.at[pl.ds(base + (nch - 1) * sub, sub)])

    return gk(table, idx)


def _knn_flat(x, k, exclude_self=False, ab_w=None, mlp_w=None):
    outs = _knn(x, k, exclude_self, ab_w, mlp_w)
    return (outs[0][:, :k].reshape(-1),) + tuple(outs[1:])


def _ec_wprep(w1, b1):
    d = w1.shape[0] // 2
    w1a, w1b = w1[:d], w1[d:]
    wcat = jnp.concatenate([w1a - w1b, w1b], axis=1)
    bcat = jnp.concatenate([b1, jnp.zeros_like(b1)])
    return wcat, bcat


def _dynconv(x, p, k1, k2, mlp_w=None, tail_mlp=None):
    n = x.shape[0]
    dh1 = p['dc1_W1'].shape[1]
    dh2 = p['dc2_W1'].shape[1]
    wcat1, bcat1 = _ec_wprep(p['dc1_W1'], p['dc1_b1'])
    wcat2, bcat2 = _ec_wprep(p['dc2_W1'], p['dc2_b1'])
    outs = _knn_flat(x, k1, ab_w=(wcat1, bcat1), mlp_w=mlp_w)
    idx1, ab1 = outs[0], outs[1]
    extra = tuple(outs[2:])
    g1 = _sc_gather(ab1[:, dh1:], idx1).reshape(n, k1 * dh1)
    h, ab2 = _ecmax_lin(ab1[:, :dh1], g1, p['dc1_W2'], p['dc1_b2'],
                        wcat2, bcat2)
    idx2 = _knn_flat(h, k2)[0]
    g2 = _sc_gather(ab2[:, dh2:], idx2).reshape(n, k2 * dh2)
    if tail_mlp is None:
        res = jnp.zeros((n, p['dc2_W2'].shape[1]), jnp.float32)
        feat = _ecmax(ab2[:, :dh2], g2, p['dc2_W2'], p['dc2_b2'], res)
    else:
        feat = _ecmax_mlp2(ab2[:, :dh2], g2, p['dc2_W2'], p['dc2_b2'],
                           *tail_mlp)
    return (feat,) + extra


def kernel(ctx_xyz, ctx_tokens, pred_tokens, mask_id, params):
    p = params
    bb, mm, pp, c = pred_tokens.shape
    n_c = bb * pp
    n_t = n_c * _UP

    pred_tok_m = jnp.take(pred_tokens, mask_id, axis=1)
    distinct = pred_tok_m.reshape(n_c, c)
    ctx_tok_f = ctx_tokens.reshape(n_c, c)
    ctx_xyz_f = ctx_xyz.reshape(n_c, 3)

    ctx_out = _dynconv(
        ctx_tok_f, p, 16, 8,
        tail_mlp=(p['cd_W1'], p['cd_b1'], p['cd_W2'], p['cd_b2'],
                  ctx_xyz_f, 0.05))[0]

    feat_t, seed = _dynconv(
        distinct, p, 4, 2,
        mlp_w=(p['lat_W1'], p['lat_b1'], p['lat_W2'], p['lat_b2']))
    tgt_feat = jnp.broadcast_to(
        feat_t.reshape(bb, pp, 1, feat_t.shape[1]),
        (bb, pp, _UP, feat_t.shape[1])).reshape(n_t, feat_t.shape[1])
    seed_rep = jnp.broadcast_to(
        seed.reshape(bb, pp, 1, 3), (bb, pp, _UP, 3)).reshape(bb, pp * _UP, 3)
    noise = jax.random.normal(jax.random.key(777), (bb, pp * _UP, 3),
                              dtype=jnp.float32) * 0.02
    tgt_xyz_f = (seed_rep + noise).reshape(n_t, 3)

    x_fold = jnp.concatenate([tgt_xyz_f, tgt_feat], axis=1)
    wcat_r, bcat_r = _ec_wprep(p['rf_W1'], p['rf_b1'])
    tgt_xyz_f, ab_r = _fo_mlp(x_fold, tgt_xyz_f, p['fo_W1'], p['fo_b1'],
                              p['fo_W2'], p['fo_b2'], p['fo_W3'], p['fo_b3'],
                              wcat_r, bcat_r)

    idx_f = _knn_flat(tgt_xyz_f, 16, exclude_self=True)[0]
    dh_r = p['rf_W1'].shape[1]
    g_r = _sc_gather(ab_r[:, dh_r:], idx_f).reshape(n_t, 16 * dh_r)
    tgt_out = _ecmax(ab_r[:, :dh_r], g_r, p['rf_W2'], p['rf_b2'], tgt_xyz_f)

    return jnp.concatenate([ctx_out, tgt_out], axis=0)
